# baseline (speedup 1.0000x reference)
import functools

import jax
import jax.numpy as jnp
from jax import lax
from jax.experimental import pallas as pl
from jax.experimental.pallas import tpu as pltpu
from jax.experimental.pallas import tpu_sc as plsc

NC = 2
NS = 16
NW = NC * NS
B = 8192
H = 128
GYD = 512
CELLS = 512 * 512
SEG = CELLS // NW
PTS = B // NW
K = 25
NCH = PTS * K // 128

_mesh = plsc.VectorSubcoreMesh(core_axis_name="c", subcore_axis_name="s")
_params = pltpu.CompilerParams(needs_layout_passes=False,
                               use_tc_tiling_on_sc=True)


def _wid():
    return lax.axis_index("s") * NC + lax.axis_index("c")


@functools.partial(
    pl.kernel,
    out_type=jax.ShapeDtypeStruct((CELLS,), jnp.int32),
    mesh=_mesh,
    compiler_params=_params,
    scratch_types=[
        pltpu.VMEM((2 * B,), jnp.int32),
        pltpu.VMEM((SEG,), jnp.int32),
    ],
)
def _build(gi_hbm, owner_hbm, giv, ownv):
    wid = _wid()
    seg0 = wid * SEG

    pltpu.sync_copy(gi_hbm, giv)

    neg1 = jnp.full((16,), -1, jnp.int32)

    def init_body(i, carry):
        ownv[pl.ds(i * 16, 16)] = neg1
        return carry

    lax.fori_loop(0, SEG // 16, init_body, 0)

    iota = lax.iota(jnp.int32, 16)

    def scan_body(v, carry):
        b0 = v * 16
        pvec = 2 * (b0 + iota)
        gxc = plsc.load_gather(giv, [pvec])
        gyc = plsc.load_gather(giv, [pvec + 1])
        flat = gxc * GYD + gyc
        _, last = plsc.scan_count(flat)
        local = flat - seg0
        inr = (local >= 0) & (local < SEG)
        lc = jnp.clip(local, 0, SEG - 1)
        plsc.store_scatter(ownv, [lc], b0 + iota, mask=last & inr)
        return carry

    lax.fori_loop(0, B // 16, scan_body, 0)

    pltpu.sync_copy(ownv, owner_hbm.at[pl.ds(seg0, SEG)])


@functools.partial(
    pl.kernel,
    out_type=jax.ShapeDtypeStruct((B * K, H), jnp.float32),
    mesh=_mesh,
    compiler_params=_params,
    scratch_types=[
        pltpu.VMEM((2 * PTS,), jnp.int32),
        pltpu.VMEM((NCH, 128), jnp.int32),
        pltpu.VMEM((NCH, 128), jnp.int32),
        pltpu.VMEM((144,), jnp.int32),
        pltpu.VMEM((144,), jnp.int32),
        pltpu.VMEM((2, 128, H), jnp.float32),
        pltpu.VMEM((16, H), jnp.float32),
        pltpu.SemaphoreType.DMA((2,)),
        pltpu.SemaphoreType.DMA((2,)),
        pltpu.SemaphoreType.DMA((2,)),
        pltpu.SemaphoreType.DMA,
    ],
)
def _gather(gi_hbm, owner_hbm, mem_hbm, upd_hbm, out_hbm,
            giv, cellidx, ownall, ppos, pown, rows, ubuf,
            semO, semR, semW, semU):
    wid = _wid()
    p0 = wid * PTS
    pltpu.sync_copy(gi_hbm.at[pl.ds(2 * p0, 2 * PTS)], giv)

    iota = lax.iota(jnp.int32, 16)

    for k in range(K):
        di = k // 5 - 2
        dj = k % 5 - 2
        for h in range(2):
            j = 2 * k + h

            def pre_body(v, carry, h=h, j=j, di=di, dj=dj):
                b0 = h * 128 + v * 16
                pvec = 2 * (b0 + iota)
                gxc = plsc.load_gather(giv, [pvec])
                gyc = plsc.load_gather(giv, [pvec + 1])
                cx = jnp.maximum(gxc + di, 0)
                cy = jnp.maximum(gyc + dj, 0)
                cellidx[j, pl.ds(v * 16, 16)] = cx * GYD + cy
                return carry

            lax.fori_loop(0, 8, pre_body, 0)

    def fire_own(j, carry):
        pltpu.async_copy(owner_hbm.at[cellidx.at[j]], ownall.at[j], semO.at[0])
        return carry

    lax.fori_loop(0, NCH, fire_own, 0)

    def issue(j, s):
        pltpu.async_copy(mem_hbm.at[cellidx.at[j]], rows.at[s], semR.at[s])

    issue(0, 0)

    def drain_own(j, carry):
        pltpu.make_async_copy(
            owner_hbm.at[pl.ds(0, 128)], ownall.at[j], semO.at[0]).wait()
        return carry

    lax.fori_loop(0, NCH, drain_own, 0)

    def chunk_body(j, carry):
        s = j % 2
        ns = 1 - s

        @pl.when(j + 1 < NCH)
        def _prefetch():
            @pl.when(j >= 1)
            def _drain_out():
                pltpu.make_async_copy(
                    rows.at[ns], out_hbm.at[pl.ds(0, 128)], semW.at[ns]).wait()

            issue(j + 1, ns)

        def cmp_body(u, cnt):
            o = ownall[j, pl.ds(u * 16, 16)]
            m = o >= 0
            plsc.store_compressed(ppos.at[pl.ds(cnt, 16)], u * 16 + iota, mask=m)
            plsc.store_compressed(pown.at[pl.ds(cnt, 16)], o, mask=m)
            return cnt + plsc.all_reduce_population_count(m)[0]

        n = lax.fori_loop(0, 8, cmp_body, 0)

        pltpu.make_async_copy(
            mem_hbm.at[pl.ds(0, 128)], rows.at[s], semR.at[s]).wait()

        @pl.when(n > 0)
        def _patch():
            pos0 = ppos[pl.ds(0, 16)][0]
            own0 = pown[pl.ds(0, 16)][0]

            def batch_body(bi, carry2):
                base = bi * 16
                posv = ppos[pl.ds(base, 16)]
                ownv2 = pown[pl.ds(base, 16)]
                valid = (base + iota) < n
                posv = jnp.where(valid, posv, pos0)
                ownv2 = jnp.where(valid, ownv2, own0)
                pltpu.async_copy(upd_hbm.at[ownv2], ubuf, semU).wait()
                for r in range(16):
                    p = posv[r]
                    for u8 in range(8):
                        rows[s, p, pl.ds(u8 * 16, 16)] = ubuf[r, pl.ds(u8 * 16, 16)]
                return carry2

            lax.fori_loop(0, (n + 15) // 16, batch_body, 0)

        out0 = (j // 2) * B + p0 + s * 128
        pltpu.async_copy(rows.at[s], out_hbm.at[pl.ds(out0, 128)], semW.at[s])
        return carry

    lax.fori_loop(0, NCH, chunk_body, 0)

    pltpu.make_async_copy(rows.at[0], out_hbm.at[pl.ds(0, 128)], semW.at[0]).wait()
    pltpu.make_async_copy(rows.at[1], out_hbm.at[pl.ds(0, 128)], semW.at[1]).wait()


def kernel(grid_input, updates, memory):
    gi = grid_input.reshape(2 * B)
    memflat = memory.reshape(CELLS, H)
    owner = _build(gi)
    outflat = _gather(gi, owner, memflat, updates)
    return outflat.reshape(B, K, H)


# speedup vs baseline: 1.8317x; 1.8317x over previous
import functools

import jax
import jax.numpy as jnp
from jax import lax
from jax.experimental import pallas as pl
from jax.experimental.pallas import tpu as pltpu
from jax.experimental.pallas import tpu_sc as plsc

NC = 2
NS = 16
NW = NC * NS
B = 8192
H = 128
GYD = 512
CELLS = 512 * 512
SEG = CELLS // NW
PTS = B // NW
K = 25
NCH = PTS * K // 128

_mesh = plsc.VectorSubcoreMesh(core_axis_name="c", subcore_axis_name="s")
_params = pltpu.CompilerParams(needs_layout_passes=False,
                               use_tc_tiling_on_sc=True)


def _wid():
    return lax.axis_index("s") * NC + lax.axis_index("c")


@functools.partial(
    pl.kernel,
    out_type=jax.ShapeDtypeStruct((CELLS,), jnp.int32),
    mesh=_mesh,
    compiler_params=_params,
    scratch_types=[
        pltpu.VMEM((2 * B,), jnp.int32),
        pltpu.VMEM((SEG,), jnp.int32),
    ],
)
def _build(gi_hbm, owner_hbm, giv, ownv):
    wid = _wid()
    seg0 = wid * SEG

    pltpu.sync_copy(gi_hbm, giv)

    neg1 = jnp.full((16,), -1, jnp.int32)

    def init_body(i, carry):
        ownv[pl.ds(i * 16, 16)] = neg1
        return carry

    lax.fori_loop(0, SEG // 16, init_body, 0)

    iota = lax.iota(jnp.int32, 16)

    def scan_body(v, carry):
        b0 = v * 16
        pvec = 2 * (b0 + iota)
        gxc = plsc.load_gather(giv, [pvec])
        gyc = plsc.load_gather(giv, [pvec + 1])
        flat = gxc * GYD + gyc
        _, last = plsc.scan_count(flat)
        local = flat - seg0
        inr = (local >= 0) & (local < SEG)
        lc = jnp.clip(local, 0, SEG - 1)
        plsc.store_scatter(ownv, [lc], b0 + iota, mask=last & inr)
        return carry

    lax.fori_loop(0, B // 16, scan_body, 0)

    pltpu.sync_copy(ownv, owner_hbm.at[pl.ds(seg0, SEG)])


@functools.partial(
    pl.kernel,
    out_type=jax.ShapeDtypeStruct((B * K, H), jnp.float32),
    mesh=_mesh,
    compiler_params=_params,
    scratch_types=[
        pltpu.VMEM((2 * B,), jnp.int32),
        pltpu.VMEM((NCH, 128), jnp.int32),
        pltpu.VMEM((NCH, 128), jnp.int32),
        pltpu.VMEM((144,), jnp.int32),
        pltpu.VMEM((144,), jnp.int32),
        pltpu.VMEM((2, 128, H), jnp.float32),
        pltpu.VMEM((16, H), jnp.float32),
        pltpu.SemaphoreType.DMA((2,)),
        pltpu.SemaphoreType.DMA((2,)),
        pltpu.SemaphoreType.DMA((2,)),
        pltpu.SemaphoreType.DMA,
    ],
)
def _gather(gi_hbm, owner_hbm, mem_hbm, upd_hbm, out_hbm,
            giv, cellidx, ownall, ppos, pown, rows, ubuf,
            semO, semR, semW, semU):
    wid = _wid()
    row0 = wid * (K * PTS)
    pltpu.sync_copy(gi_hbm, giv)

    iota = lax.iota(jnp.int32, 16)

    def pre_body(t, carry):
        p = row0 + t * 16 + iota
        n = 25 * (p & (B - 1)) + (p >> 13)
        k0 = n >> 13
        b0 = n & (B - 1)
        q = k0 // 5
        di = q - 2
        dj = k0 - 5 * q - 2
        gxc = plsc.load_gather(giv, [2 * b0])
        gyc = plsc.load_gather(giv, [2 * b0 + 1])
        cx = jnp.maximum(gxc + di, 0)
        cy = jnp.maximum(gyc + dj, 0)
        cellidx[t >> 3, pl.ds((t & 7) * 16, 16)] = cx * GYD + cy
        return carry

    lax.fori_loop(0, K * PTS // 16, pre_body, 0)

    def fire_own(j, carry):
        pltpu.async_copy(owner_hbm.at[cellidx.at[j]], ownall.at[j], semO.at[0])
        return carry

    lax.fori_loop(0, NCH, fire_own, 0)

    def issue(j, s):
        pltpu.async_copy(mem_hbm.at[cellidx.at[j]], rows.at[s], semR.at[s])

    issue(0, 0)

    def drain_own(j, carry):
        pltpu.make_async_copy(
            owner_hbm.at[pl.ds(0, 128)], ownall.at[j], semO.at[0]).wait()
        return carry

    lax.fori_loop(0, NCH, drain_own, 0)

    def chunk_body(j, carry):
        s = j % 2
        ns = 1 - s

        @pl.when(j + 1 < NCH)
        def _prefetch():
            @pl.when(j >= 1)
            def _drain_out():
                pltpu.make_async_copy(
                    rows.at[ns], out_hbm.at[pl.ds(0, 128)], semW.at[ns]).wait()

            issue(j + 1, ns)

        def cmp_body(u, cnt):
            o = ownall[j, pl.ds(u * 16, 16)]
            m = o >= 0
            plsc.store_compressed(ppos.at[pl.ds(cnt, 16)], u * 16 + iota, mask=m)
            plsc.store_compressed(pown.at[pl.ds(cnt, 16)], o, mask=m)
            return cnt + plsc.all_reduce_population_count(m)[0]

        n = lax.fori_loop(0, 8, cmp_body, 0)

        pltpu.make_async_copy(
            mem_hbm.at[pl.ds(0, 128)], rows.at[s], semR.at[s]).wait()

        @pl.when(n > 0)
        def _patch():
            pos0 = ppos[pl.ds(0, 16)][0]
            own0 = pown[pl.ds(0, 16)][0]

            def batch_body(bi, carry2):
                base = bi * 16
                posv = ppos[pl.ds(base, 16)]
                ownv2 = pown[pl.ds(base, 16)]
                valid = (base + iota) < n
                posv = jnp.where(valid, posv, pos0)
                ownv2 = jnp.where(valid, ownv2, own0)
                pltpu.async_copy(upd_hbm.at[ownv2], ubuf, semU).wait()
                for r in range(16):
                    p = posv[r]
                    for u8 in range(8):
                        rows[s, p, pl.ds(u8 * 16, 16)] = ubuf[r, pl.ds(u8 * 16, 16)]
                return carry2

            lax.fori_loop(0, (n + 15) // 16, batch_body, 0)

        out0 = row0 + j * 128
        pltpu.async_copy(rows.at[s], out_hbm.at[pl.ds(out0, 128)], semW.at[s])
        return carry

    lax.fori_loop(0, NCH, chunk_body, 0)

    pltpu.make_async_copy(rows.at[0], out_hbm.at[pl.ds(0, 128)], semW.at[0]).wait()
    pltpu.make_async_copy(rows.at[1], out_hbm.at[pl.ds(0, 128)], semW.at[1]).wait()


def kernel(grid_input, updates, memory):
    gi = grid_input.reshape(2 * B)
    memflat = memory.reshape(CELLS, H)
    owner = _build(gi)
    outflat = _gather(gi, owner, memflat, updates)
    return outflat.reshape(K, B, H).transpose(1, 0, 2)


# speedup vs baseline: 2.7108x; 1.4799x over previous
import functools

import jax
import jax.numpy as jnp
from jax import lax
from jax.experimental import pallas as pl
from jax.experimental.pallas import tpu as pltpu
from jax.experimental.pallas import tpu_sc as plsc

NC = 2
NS = 16
NW = NC * NS
B = 8192
H = 128
GYD = 512
CELLS = 512 * 512
SEG = CELLS // NW
PTS = B // NW
K = 25
NCH = PTS * K // 128
PB = 64
PBN = PTS * K // PB + 2

_mesh = plsc.VectorSubcoreMesh(core_axis_name="c", subcore_axis_name="s")
_params = pltpu.CompilerParams(needs_layout_passes=False,
                               use_tc_tiling_on_sc=True)


def _wid():
    return lax.axis_index("s") * NC + lax.axis_index("c")


@functools.partial(
    pl.kernel,
    out_type=jax.ShapeDtypeStruct((CELLS,), jnp.int32),
    mesh=_mesh,
    compiler_params=_params,
    scratch_types=[
        pltpu.VMEM((2 * B,), jnp.int32),
        pltpu.VMEM((SEG,), jnp.int32),
    ],
)
def _build(gi_hbm, owner_hbm, giv, ownv):
    wid = _wid()
    seg0 = wid * SEG

    pltpu.sync_copy(gi_hbm, giv)

    neg1 = jnp.full((16,), -1, jnp.int32)

    def init_body(i, carry):
        ownv[pl.ds(i * 16, 16)] = neg1
        return carry

    lax.fori_loop(0, SEG // 16, init_body, 0)

    iota = lax.iota(jnp.int32, 16)

    def scan_body(v, carry):
        b0 = v * 16
        pvec = 2 * (b0 + iota)
        gxc = plsc.load_gather(giv, [pvec])
        gyc = plsc.load_gather(giv, [pvec + 1])
        flat = gxc * GYD + gyc
        _, last = plsc.scan_count(flat)
        local = flat - seg0
        inr = (local >= 0) & (local < SEG)
        lc = jnp.clip(local, 0, SEG - 1)
        plsc.store_scatter(ownv, [lc], b0 + iota, mask=last & inr)
        return carry

    lax.fori_loop(0, B // 16, scan_body, 0)

    pltpu.sync_copy(ownv, owner_hbm.at[pl.ds(seg0, SEG)])


@functools.partial(
    pl.kernel,
    out_type=jax.ShapeDtypeStruct((B * K, H), jnp.float32),
    mesh=_mesh,
    compiler_params=_params,
    scratch_types=[
        pltpu.VMEM((2 * B,), jnp.int32),
        pltpu.VMEM((NCH, 128), jnp.int32),
        pltpu.VMEM((NCH, 128), jnp.int32),
        pltpu.VMEM((PBN, PB), jnp.int32),
        pltpu.VMEM((PBN, PB), jnp.int32),
        pltpu.VMEM((3, 128, H), jnp.float32),
        pltpu.VMEM((2, PB, H), jnp.float32),
        pltpu.SemaphoreType.DMA((2,)),
        pltpu.SemaphoreType.DMA((3,)),
        pltpu.SemaphoreType.DMA((3,)),
        pltpu.SemaphoreType.DMA,
        pltpu.SemaphoreType.DMA((2,)),
    ],
)
def _gather(gi_hbm, owner_hbm, mem_hbm, upd_hbm, out_hbm,
            giv, cellidx, ownall, pidx, pown, rows, ubuf,
            semO, semR, semW, semU, semP):
    wid = _wid()
    row0 = wid * (K * PTS)
    pltpu.sync_copy(gi_hbm, giv)

    iota = lax.iota(jnp.int32, 16)

    def pre_body(t, carry):
        p = row0 + t * 16 + iota
        n = 25 * (p & (B - 1)) + (p >> 13)
        k0 = n >> 13
        b0 = n & (B - 1)
        q = k0 // 5
        di = q - 2
        dj = k0 - 5 * q - 2
        gxc = plsc.load_gather(giv, [2 * b0])
        gyc = plsc.load_gather(giv, [2 * b0 + 1])
        cx = jnp.maximum(gxc + di, 0)
        cy = jnp.maximum(gyc + dj, 0)
        cellidx[t >> 3, pl.ds((t & 7) * 16, 16)] = cx * GYD + cy
        return carry

    lax.fori_loop(0, K * PTS // 16, pre_body, 0)

    def fire_own(j, carry):
        pltpu.async_copy(owner_hbm.at[cellidx.at[j]], ownall.at[j], semO.at[0])
        return carry

    lax.fori_loop(0, NCH, fire_own, 0)

    def issue(j):
        s = j % 3
        pltpu.async_copy(mem_hbm.at[cellidx.at[j]], rows.at[s], semR.at[s])

    issue(0)
    issue(1)

    def drain_own(j, carry):
        pltpu.make_async_copy(
            owner_hbm.at[pl.ds(0, 128)], ownall.at[j], semO.at[0]).wait()
        return carry

    lax.fori_loop(0, NCH, drain_own, 0)

    def cmp_body(t, cnt):
        o = ownall[t >> 3, pl.ds((t & 7) * 16, 16)]
        m = o >= 0
        dest = cnt + plsc.cumsum(m.astype(jnp.int32)) - 1
        orow = row0 + t * 16 + iota
        plsc.store_scatter(pidx, [dest >> 6, dest & (PB - 1)], orow, mask=m)
        plsc.store_scatter(pown, [dest >> 6, dest & (PB - 1)], o, mask=m)
        return cnt + plsc.all_reduce_population_count(m)[0]

    n = lax.fori_loop(0, K * PTS // 16, cmp_body, 0)

    def chunk_body(j, carry):
        s = j % 3

        pltpu.make_async_copy(
            mem_hbm.at[pl.ds(0, 128)], rows.at[s], semR.at[s]).wait()
        out0 = row0 + j * 128
        pltpu.async_copy(rows.at[s], out_hbm.at[pl.ds(out0, 128)], semW.at[s])

        @pl.when(j + 2 < NCH)
        def _prefetch():
            s2 = (j + 2) % 3

            @pl.when(j >= 1)
            def _drain_out():
                pltpu.make_async_copy(
                    rows.at[s2], out_hbm.at[pl.ds(0, 128)], semW.at[s2]).wait()

            issue(j + 2)

        return carry

    lax.fori_loop(0, NCH, chunk_body, 0)

    for jlast in (NCH - 2, NCH - 1):
        s2 = jlast % 3
        pltpu.make_async_copy(
            rows.at[s2], out_hbm.at[pl.ds(0, 128)], semW.at[s2]).wait()

    @pl.when(n > 0)
    def _patch():
        pos0 = pidx[0, pl.ds(0, 16)][0]
        own0 = pown[0, pl.ds(0, 16)][0]

        def pad_body(v, carry):
            dest = n + v * 16 + iota
            m = dest < PBN * PB
            dc = jnp.minimum(dest, PBN * PB - 1)
            plsc.store_scatter(pidx, [dc >> 6, dc & (PB - 1)],
                               jnp.zeros((16,), jnp.int32) + pos0, mask=m)
            plsc.store_scatter(pown, [dc >> 6, dc & (PB - 1)],
                               jnp.zeros((16,), jnp.int32) + own0, mask=m)
            return carry

        lax.fori_loop(0, PB // 16, pad_body, 0)

        nb = (n + PB - 1) // PB

        def patch_body(bi, carry):
            sl = bi % 2

            @pl.when(bi >= 2)
            def _drain_scatter():
                pltpu.make_async_copy(
                    ubuf.at[sl], out_hbm.at[pl.ds(0, PB)], semP.at[sl]).wait()

            pltpu.async_copy(upd_hbm.at[pown.at[bi]], ubuf.at[sl], semU).wait()
            pltpu.async_copy(ubuf.at[sl], out_hbm.at[pidx.at[bi]], semP.at[sl])
            return carry

        lax.fori_loop(0, nb, patch_body, 0)

        def drain_body(bi, carry):
            pltpu.make_async_copy(
                ubuf.at[bi % 2], out_hbm.at[pl.ds(0, PB)], semP.at[bi % 2]).wait()
            return carry

        lax.fori_loop(jnp.maximum(nb - 2, 0), nb, drain_body, 0)


def kernel(grid_input, updates, memory):
    gi = grid_input.reshape(2 * B)
    memflat = memory.reshape(CELLS, H)
    owner = _build(gi)
    outflat = _gather(gi, owner, memflat, updates)
    return outflat.reshape(K, B, H).transpose(1, 0, 2)


# speedup vs baseline: 2.9263x; 1.0795x over previous
import functools

import jax
import jax.numpy as jnp
from jax import lax
from jax.experimental import pallas as pl
from jax.experimental.pallas import tpu as pltpu
from jax.experimental.pallas import tpu_sc as plsc

NC = 2
NS = 16
NW = NC * NS
B = 8192
H = 128
GYD = 512
CELLS = 512 * 512
SEG = CELLS // NW
PTS = B // NW
K = 25
NCH = PTS * K // 128
PB = 64
PBN = PTS * K // PB + 2

_mesh = plsc.VectorSubcoreMesh(core_axis_name="c", subcore_axis_name="s")
_params = pltpu.CompilerParams(needs_layout_passes=False,
                               use_tc_tiling_on_sc=True)


def _wid():
    return lax.axis_index("s") * NC + lax.axis_index("c")


@functools.partial(
    pl.kernel,
    out_type=jax.ShapeDtypeStruct((CELLS,), jnp.int32),
    mesh=_mesh,
    compiler_params=_params,
    scratch_types=[
        pltpu.VMEM((2 * B,), jnp.int32),
        pltpu.VMEM((SEG,), jnp.int32),
    ],
)
def _build(gi_hbm, owner_hbm, giv, ownv):
    wid = _wid()
    seg0 = wid * SEG

    pltpu.sync_copy(gi_hbm, giv)

    neg1 = jnp.full((16,), -1, jnp.int32)

    def init_body(i, carry):
        ownv[pl.ds(i * 16, 16)] = neg1
        return carry

    lax.fori_loop(0, SEG // 16, init_body, 0)

    iota = lax.iota(jnp.int32, 16)

    def scan_body(v, carry):
        b0 = v * 16
        pvec = 2 * (b0 + iota)
        gxc = plsc.load_gather(giv, [pvec])
        gyc = plsc.load_gather(giv, [pvec + 1])
        flat = gxc * GYD + gyc
        _, last = plsc.scan_count(flat)
        local = flat - seg0
        inr = (local >= 0) & (local < SEG)
        lc = jnp.clip(local, 0, SEG - 1)
        plsc.store_scatter(ownv, [lc], b0 + iota, mask=last & inr)
        return carry

    lax.fori_loop(0, B // 16, scan_body, 0)

    pltpu.sync_copy(ownv, owner_hbm.at[pl.ds(seg0, SEG)])


@functools.partial(
    pl.kernel,
    out_type=jax.ShapeDtypeStruct((B * K, H), jnp.float32),
    mesh=_mesh,
    compiler_params=_params,
    scratch_types=[
        pltpu.VMEM((2 * B,), jnp.int32),
        pltpu.VMEM((NCH, 128), jnp.int32),
        pltpu.VMEM((8, 128), jnp.int32),
        pltpu.VMEM((PBN, PB), jnp.int32),
        pltpu.VMEM((PBN, PB), jnp.int32),
        pltpu.VMEM((3, 128, H), jnp.float32),
        pltpu.VMEM((2, PB, H), jnp.float32),
        pltpu.SemaphoreType.DMA((8,)),
        pltpu.SemaphoreType.DMA((3,)),
        pltpu.SemaphoreType.DMA((3,)),
        pltpu.SemaphoreType.DMA,
        pltpu.SemaphoreType.DMA((2,)),
    ],
)
def _gather(gi_hbm, owner_hbm, mem_hbm, upd_hbm, out_hbm,
            giv, cellidx, ownb, pidx, pown, rows, ubuf,
            semO, semR, semW, semU, semP):
    wid = _wid()
    row0 = wid * (K * PTS)
    pltpu.sync_copy(gi_hbm, giv)

    iota = lax.iota(jnp.int32, 16)

    def pre_body(t, carry):
        p = row0 + t * 16 + iota
        n = 25 * (p & (B - 1)) + (p >> 13)
        k0 = n >> 13
        b0 = n & (B - 1)
        q = k0 // 5
        di = q - 2
        dj = k0 - 5 * q - 2
        gxc = plsc.load_gather(giv, [2 * b0])
        gyc = plsc.load_gather(giv, [2 * b0 + 1])
        cx = jnp.maximum(gxc + di, 0)
        cy = jnp.maximum(gyc + dj, 0)
        cellidx[t >> 3, pl.ds((t & 7) * 16, 16)] = cx * GYD + cy
        return carry

    lax.fori_loop(0, K * PTS // 16, pre_body, 0)

    def fire_own(j):
        sl = j % 8
        pltpu.async_copy(owner_hbm.at[cellidx.at[j]], ownb.at[sl], semO.at[sl])

    for j0 in range(8):
        fire_own(j0)

    def issue(j):
        s = j % 3
        pltpu.async_copy(mem_hbm.at[cellidx.at[j]], rows.at[s], semR.at[s])

    issue(0)
    issue(1)

    def chunk_body(j, cnt):
        s = j % 3
        sl = j % 8

        pltpu.make_async_copy(
            owner_hbm.at[pl.ds(0, 128)], ownb.at[sl], semO.at[sl]).wait()
        for u in range(8):
            o = ownb[sl, pl.ds(u * 16, 16)]
            m = o >= 0
            dest = cnt + plsc.cumsum(m.astype(jnp.int32)) - 1
            orow = row0 + j * 128 + u * 16 + iota
            plsc.store_scatter(pidx, [dest >> 6, dest & (PB - 1)], orow,
                               mask=m)
            plsc.store_scatter(pown, [dest >> 6, dest & (PB - 1)], o, mask=m)
            cnt = cnt + plsc.all_reduce_population_count(m)[0]

        @pl.when(j + 8 < NCH)
        def _refire():
            fire_own(j + 8)

        pltpu.make_async_copy(
            mem_hbm.at[pl.ds(0, 128)], rows.at[s], semR.at[s]).wait()
        out0 = row0 + j * 128
        pltpu.async_copy(rows.at[s], out_hbm.at[pl.ds(out0, 128)], semW.at[s])

        @pl.when(j + 2 < NCH)
        def _prefetch():
            s2 = (j + 2) % 3

            @pl.when(j >= 1)
            def _drain_out():
                pltpu.make_async_copy(
                    rows.at[s2], out_hbm.at[pl.ds(0, 128)], semW.at[s2]).wait()

            issue(j + 2)

        return cnt

    n = lax.fori_loop(0, NCH, chunk_body, 0)

    for jlast in (NCH - 2, NCH - 1):
        s2 = jlast % 3
        pltpu.make_async_copy(
            rows.at[s2], out_hbm.at[pl.ds(0, 128)], semW.at[s2]).wait()

    @pl.when(n > 0)
    def _patch():
        pos0 = pidx[0, pl.ds(0, 16)][0]
        own0 = pown[0, pl.ds(0, 16)][0]

        def pad_body(v, carry):
            dest = n + v * 16 + iota
            m = dest < PBN * PB
            dc = jnp.minimum(dest, PBN * PB - 1)
            plsc.store_scatter(pidx, [dc >> 6, dc & (PB - 1)],
                               jnp.zeros((16,), jnp.int32) + pos0, mask=m)
            plsc.store_scatter(pown, [dc >> 6, dc & (PB - 1)],
                               jnp.zeros((16,), jnp.int32) + own0, mask=m)
            return carry

        lax.fori_loop(0, PB // 16, pad_body, 0)

        nb = (n + PB - 1) // PB

        def patch_body(bi, carry):
            sl = bi % 2

            @pl.when(bi >= 2)
            def _drain_scatter():
                pltpu.make_async_copy(
                    ubuf.at[sl], out_hbm.at[pl.ds(0, PB)], semP.at[sl]).wait()

            pltpu.async_copy(upd_hbm.at[pown.at[bi]], ubuf.at[sl], semU).wait()
            pltpu.async_copy(ubuf.at[sl], out_hbm.at[pidx.at[bi]], semP.at[sl])
            return carry

        lax.fori_loop(0, nb, patch_body, 0)

        def drain_body(bi, carry):
            pltpu.make_async_copy(
                ubuf.at[bi % 2], out_hbm.at[pl.ds(0, PB)], semP.at[bi % 2]).wait()
            return carry

        lax.fori_loop(jnp.maximum(nb - 2, 0), nb, drain_body, 0)


def kernel(grid_input, updates, memory):
    gi = grid_input.reshape(2 * B)
    memflat = memory.reshape(CELLS, H)
    owner = _build(gi)
    outflat = _gather(gi, owner, memflat, updates)
    return outflat.reshape(K, B, H).transpose(1, 0, 2)
